# baseline (device time: 138886 ns/iter reference)
import jax
import jax.numpy as jnp
import numpy as np
from jax import lax
from jax.experimental import pallas as pl
from jax.experimental.pallas import tpu as pltpu

N_DEV = 4
SQ = 2048
D_MODEL = 1024
HQ_SH = 8
DH = 128
BQ = 256
NBQ = SQ // BQ
KW = 512
HR = SQ // 4
QR = SQ // 8
BASE_B = SQ // 2
SCALE = 0.08838834764831843


def _fused(x2, Wq, K_sh, V_sh, Wo):
    def body(x_ref, wq_ref, k_ref, v_ref, wo_ref, out_ref,
             c1a, c1b, c2a, c2b, sa_send, sa_recv, sb_send, sb_recv):
        me = lax.axis_index("i")
        py = me ^ 1
        px = 3 - me

        barrier = pltpu.get_barrier_semaphore()
        for nbr in (py, px):
            pl.semaphore_signal(
                barrier, inc=1, device_id=(nbr,),
                device_id_type=pl.DeviceIdType.MESH,
            )
        pl.semaphore_wait(barrier, 2)

        bf16 = jnp.bfloat16
        wq_bf = wq_ref[...].astype(bf16)
        wo_bf = wo_ref[...].astype(bf16)

        def compute_block(row0):
            start = jnp.clip(row0 - 128, 0, SQ - KW)
            q = jnp.dot(x_ref[pl.ds(row0, BQ), :].astype(bf16), wq_bf,
                        preferred_element_type=jnp.float32)
            q_glob = row0 + lax.broadcasted_iota(jnp.int32, (BQ, KW), 0)
            k_glob = start + lax.broadcasted_iota(jnp.int32, (BQ, KW), 1)
            mask = jnp.abs(q_glob - k_glob) <= 128
            ctx_heads = []
            for h in range(HQ_SH):
                qh = q[:, h * DH:(h + 1) * DH].astype(bf16)
                kh = k_ref[h, pl.ds(start, KW), :].astype(bf16)
                vh = v_ref[h, pl.ds(start, KW), :].astype(bf16)
                s = lax.dot_general(
                    qh, kh, (((1,), (1,)), ((), ())),
                    preferred_element_type=jnp.float32) * SCALE
                s = jnp.where(mask, s, -1e9)
                m = s.max(axis=1, keepdims=True)
                w = jnp.exp(s - m)
                w = w / w.sum(axis=1, keepdims=True)
                ctx_heads.append(jnp.dot(w.astype(bf16), vh,
                                         preferred_element_type=jnp.float32))
            ctx = jnp.concatenate(ctx_heads, axis=1)
            out_ref[pl.ds(row0, BQ), :] = jnp.dot(
                ctx.astype(bf16), wo_bf, preferred_element_type=jnp.float32)

        fa = (me ^ (me >> 1)) & 1
        ga = (me >> 1) & 1
        fb = (me >> 1) & 1
        gb = me & 1

        def xchg(src_rows, n_rows, dst, dst_rows, sems, phase, dev):
            dst_ref = (dst if dst_rows is None
                       else dst.at[pl.ds(dst_rows, n_rows), :])
            return pltpu.make_async_remote_copy(
                src_ref=out_ref.at[pl.ds(src_rows, n_rows), :],
                dst_ref=dst_ref,
                send_sem=sems[0].at[phase],
                recv_sem=sems[1].at[phase],
                device_id=(dev,),
                device_id_type=pl.DeviceIdType.MESH,
            )

        sa = (sa_send, sa_recv)
        sb = (sb_send, sb_recv)

        a_send_r = (1 - fa) * HR
        b_send_r = BASE_B + (1 - fb) * HR
        a_keep_r = fa * HR
        b_keep_r = BASE_B + fb * HR
        for j in range(2):
            compute_block(a_send_r + j * BQ)
        for j in range(2):
            compute_block(b_send_r + j * BQ)
        a = xchg(a_send_r, HR, c1a, None, sa, 0, py)
        b = xchg(b_send_r, HR, c1b, None, sb, 0, px)
        a.start()
        b.start()
        for j in range(2):
            compute_block(a_keep_r + j * BQ)
        for j in range(2):
            compute_block(b_keep_r + j * BQ)
        a.wait()
        b.wait()
        out_ref[pl.ds(a_keep_r, HR), :] += c1a[...]
        out_ref[pl.ds(b_keep_r, HR), :] += c1b[...]

        qka = 2 * fa + ga
        qsa = 2 * fa + (1 - ga)
        qkb = 2 * fb + gb
        qsb = 2 * fb + (1 - gb)

        a = xchg(qsa * QR, QR, c2a, None, sa, 1, px)
        b = xchg(BASE_B + qsb * QR, QR, c2b, None, sb, 1, py)
        a.start(); b.start(); a.wait(); b.wait()
        out_ref[pl.ds(qka * QR, QR), :] += c2a[...]
        out_ref[pl.ds(BASE_B + qkb * QR, QR), :] += c2b[...]

        a = xchg(qka * QR, QR, out_ref, qka * QR, sa, 2, px)
        b = xchg(BASE_B + qkb * QR, QR, out_ref, BASE_B + qkb * QR, sb, 2, py)
        a.start(); b.start(); a.wait(); b.wait()

        a = xchg(fa * HR, HR, out_ref, fa * HR, sa, 3, py)
        b = xchg(BASE_B + fb * HR, HR, out_ref, BASE_B + fb * HR, sb, 3, px)
        a.start(); b.start(); a.wait(); b.wait()

    return pl.pallas_call(
        body,
        out_shape=jax.ShapeDtypeStruct((SQ, D_MODEL), jnp.float32),
        in_specs=[pl.BlockSpec(memory_space=pltpu.VMEM)] * 5,
        out_specs=pl.BlockSpec(memory_space=pltpu.VMEM),
        scratch_shapes=[
            pltpu.VMEM((HR, D_MODEL), jnp.float32),
            pltpu.VMEM((HR, D_MODEL), jnp.float32),
            pltpu.VMEM((QR, D_MODEL), jnp.float32),
            pltpu.VMEM((QR, D_MODEL), jnp.float32),
            pltpu.SemaphoreType.DMA((4,)),
            pltpu.SemaphoreType.DMA((4,)),
            pltpu.SemaphoreType.DMA((4,)),
            pltpu.SemaphoreType.DMA((4,)),
        ],
        compiler_params=pltpu.CompilerParams(
            collective_id=0,
            vmem_limit_bytes=100 * 1024 * 1024,
        ),
    )(x2, Wq, K_sh, V_sh, Wo)


def kernel(x, Wq, K_ext, V_ext, Wo):
    me = lax.axis_index("i")

    x2 = x.reshape(SQ, D_MODEL)
    K_sh = lax.dynamic_slice_in_dim(
        K_ext.reshape(SQ, 32, DH), me * HQ_SH, HQ_SH, axis=1).transpose(1, 0, 2)
    V_sh = lax.dynamic_slice_in_dim(
        V_ext.reshape(SQ, 32, DH), me * HQ_SH, HQ_SH, axis=1).transpose(1, 0, 2)

    out = _fused(x2, Wq, K_sh, V_sh, Wo)
    return out.reshape(1, SQ, D_MODEL)


# device time: 112597 ns/iter; 1.2335x vs baseline; 1.2335x over previous
import jax
import jax.numpy as jnp
import numpy as np
from jax import lax
from jax.experimental import pallas as pl
from jax.experimental.pallas import tpu as pltpu

N_DEV = 4
SQ = 2048
D_MODEL = 1024
HQ_SH = 8
DH = 128
BQ = 256
NBQ = SQ // BQ
KW = 512
HR = SQ // 4
QR = SQ // 8
BASE_B = SQ // 2
SCALE = 0.08838834764831843


def _fused(x2, Wq, K_sh, V_sh, Wo):
    def body(x_ref, wq_ref, k_hbm, v_hbm, wo_ref, out_ref,
             k_ref, v_ref, kv_sems,
             c1a, c1b, c2a, c2b, sa_send, sa_recv, sb_send, sb_recv):
        me = lax.axis_index("i")
        py = me ^ 1
        px = 3 - me

        kv_copies = []
        for h in range(HQ_SH):
            for j, (src, dst) in enumerate(((k_hbm, k_ref), (v_hbm, v_ref))):
                c = pltpu.make_async_copy(
                    src.at[:, me * HQ_SH + h, :],
                    dst.at[h],
                    kv_sems.at[2 * h + j],
                )
                c.start()
                kv_copies.append(c)

        barrier = pltpu.get_barrier_semaphore()
        for nbr in (py, px):
            pl.semaphore_signal(
                barrier, inc=1, device_id=(nbr,),
                device_id_type=pl.DeviceIdType.MESH,
            )
        pl.semaphore_wait(barrier, 2)

        for c in kv_copies:
            c.wait()

        def compute_block(row0):
            start = jnp.clip(row0 - 128, 0, SQ - KW)
            q = jnp.dot(x_ref[pl.ds(row0, BQ), :], wq_ref[...],
                        preferred_element_type=jnp.float32)
            q_glob = row0 + lax.broadcasted_iota(jnp.int32, (BQ, KW), 0)
            k_glob = start + lax.broadcasted_iota(jnp.int32, (BQ, KW), 1)
            mask = jnp.abs(q_glob - k_glob) <= 128
            ctx_heads = []
            for h in range(HQ_SH):
                qh = q[:, h * DH:(h + 1) * DH]
                kh = k_ref[h, pl.ds(start, KW), :]
                vh = v_ref[h, pl.ds(start, KW), :]
                s = lax.dot_general(
                    qh, kh, (((1,), (1,)), ((), ())),
                    preferred_element_type=jnp.float32) * SCALE
                s = jnp.where(mask, s, -1e9)
                m = s.max(axis=1, keepdims=True)
                w = jnp.exp(s - m)
                w = w / w.sum(axis=1, keepdims=True)
                ctx_heads.append(jnp.dot(w, vh,
                                         preferred_element_type=jnp.float32))
            ctx = jnp.concatenate(ctx_heads, axis=1)
            out_ref[pl.ds(row0, BQ), :] = jnp.dot(
                ctx, wo_ref[...], preferred_element_type=jnp.float32)

        fa = (me ^ (me >> 1)) & 1
        ga = (me >> 1) & 1
        fb = (me >> 1) & 1
        gb = me & 1

        def xchg(src_rows, n_rows, dst, dst_rows, sems, phase, dev):
            dst_ref = (dst if dst_rows is None
                       else dst.at[pl.ds(dst_rows, n_rows), :])
            return pltpu.make_async_remote_copy(
                src_ref=out_ref.at[pl.ds(src_rows, n_rows), :],
                dst_ref=dst_ref,
                send_sem=sems[0].at[phase],
                recv_sem=sems[1].at[phase],
                device_id=(dev,),
                device_id_type=pl.DeviceIdType.MESH,
            )

        sa = (sa_send, sa_recv)
        sb = (sb_send, sb_recv)

        a_send_r = (1 - fa) * HR
        b_send_r = BASE_B + (1 - fb) * HR
        a_keep_r = fa * HR
        b_keep_r = BASE_B + fb * HR
        for j in range(2):
            compute_block(a_send_r + j * BQ)
        for j in range(2):
            compute_block(b_send_r + j * BQ)
        a = xchg(a_send_r, HR, c1a, None, sa, 0, py)
        b = xchg(b_send_r, HR, c1b, None, sb, 0, px)
        a.start()
        b.start()
        for j in range(2):
            compute_block(a_keep_r + j * BQ)
        for j in range(2):
            compute_block(b_keep_r + j * BQ)
        a.wait()
        b.wait()
        out_ref[pl.ds(a_keep_r, HR), :] += c1a[...]
        out_ref[pl.ds(b_keep_r, HR), :] += c1b[...]

        qka = 2 * fa + ga
        qsa = 2 * fa + (1 - ga)
        qkb = 2 * fb + gb
        qsb = 2 * fb + (1 - gb)

        a = xchg(qsa * QR, QR, c2a, None, sa, 1, px)
        b = xchg(BASE_B + qsb * QR, QR, c2b, None, sb, 1, py)
        a.start(); b.start(); a.wait(); b.wait()
        out_ref[pl.ds(qka * QR, QR), :] += c2a[...]
        out_ref[pl.ds(BASE_B + qkb * QR, QR), :] += c2b[...]

        a = xchg(qka * QR, QR, out_ref, qka * QR, sa, 2, px)
        b = xchg(BASE_B + qkb * QR, QR, out_ref, BASE_B + qkb * QR, sb, 2, py)
        a.start(); b.start(); a.wait(); b.wait()

        a = xchg(fa * HR, HR, out_ref, fa * HR, sa, 3, py)
        b = xchg(BASE_B + fb * HR, HR, out_ref, BASE_B + fb * HR, sb, 3, px)
        a.start(); b.start(); a.wait(); b.wait()

    return pl.pallas_call(
        body,
        out_shape=jax.ShapeDtypeStruct((SQ, D_MODEL), jnp.float32),
        in_specs=[
            pl.BlockSpec(memory_space=pltpu.VMEM),
            pl.BlockSpec(memory_space=pltpu.VMEM),
            pl.BlockSpec(memory_space=pltpu.MemorySpace.HBM),
            pl.BlockSpec(memory_space=pltpu.MemorySpace.HBM),
            pl.BlockSpec(memory_space=pltpu.VMEM),
        ],
        out_specs=pl.BlockSpec(memory_space=pltpu.VMEM),
        scratch_shapes=[
            pltpu.VMEM((HQ_SH, SQ, DH), jnp.float32),
            pltpu.VMEM((HQ_SH, SQ, DH), jnp.float32),
            pltpu.SemaphoreType.DMA((2 * HQ_SH,)),
            pltpu.VMEM((HR, D_MODEL), jnp.float32),
            pltpu.VMEM((HR, D_MODEL), jnp.float32),
            pltpu.VMEM((QR, D_MODEL), jnp.float32),
            pltpu.VMEM((QR, D_MODEL), jnp.float32),
            pltpu.SemaphoreType.DMA((4,)),
            pltpu.SemaphoreType.DMA((4,)),
            pltpu.SemaphoreType.DMA((4,)),
            pltpu.SemaphoreType.DMA((4,)),
        ],
        compiler_params=pltpu.CompilerParams(
            collective_id=0,
            vmem_limit_bytes=100 * 1024 * 1024,
        ),
    )(x2, Wq, K_sh, V_sh, Wo)


def kernel(x, Wq, K_ext, V_ext, Wo):
    x2 = x.reshape(SQ, D_MODEL)
    out = _fused(x2, Wq, K_ext.reshape(SQ, 32, DH), V_ext.reshape(SQ, 32, DH),
                 Wo)
    return out.reshape(1, SQ, D_MODEL)


# device time: 103689 ns/iter; 1.3394x vs baseline; 1.0859x over previous
import jax
import jax.numpy as jnp
import numpy as np
from jax import lax
from jax.experimental import pallas as pl
from jax.experimental.pallas import tpu as pltpu

N_DEV = 4
SQ = 2048
D_MODEL = 1024
HQ_SH = 8
DH = 128
BQ = 256
NBQ = SQ // BQ
KW = 512
HR = SQ // 4
QR = SQ // 8
BASE_B = SQ // 2
SCALE = 0.08838834764831843


def _fused(x2, Wq, K_sh, V_sh, Wo):
    def body(x_ref, wq_ref, k_hbm, v_hbm, wo_ref, out_ref,
             k_ref, v_ref, kv_sems,
             c1a, c1b, c2a, c2b, sa_send, sa_recv, sb_send, sb_recv):
        me = lax.axis_index("i")
        py = me ^ 1
        px = 3 - me

        kv_copies = []
        for h in range(HQ_SH):
            for j, (src, dst) in enumerate(((k_hbm, k_ref), (v_hbm, v_ref))):
                c = pltpu.make_async_copy(
                    src.at[:, me * HQ_SH + h, :],
                    dst.at[h],
                    kv_sems.at[2 * h + j],
                )
                c.start()
                kv_copies.append(c)

        barrier = pltpu.get_barrier_semaphore()
        for nbr in (py, px):
            pl.semaphore_signal(
                barrier, inc=1, device_id=(nbr,),
                device_id_type=pl.DeviceIdType.MESH,
            )
        pl.semaphore_wait(barrier, 2)

        for c in kv_copies:
            c.wait()

        def compute_block(row0):
            start = jnp.clip(row0 - 128, 0, SQ - KW)
            q = jnp.dot(x_ref[pl.ds(row0, BQ), :], wq_ref[...],
                        preferred_element_type=jnp.float32)
            q_glob = row0 + lax.broadcasted_iota(jnp.int32, (BQ, KW), 0)
            k_glob = start + lax.broadcasted_iota(jnp.int32, (BQ, KW), 1)
            mask = jnp.abs(q_glob - k_glob) <= 128
            ctx_heads = []
            for h in range(HQ_SH):
                qh = q[:, h * DH:(h + 1) * DH]
                kh = k_ref[h, pl.ds(start, KW), :]
                vh = v_ref[h, pl.ds(start, KW), :]
                s = lax.dot_general(
                    qh, kh, (((1,), (1,)), ((), ())),
                    preferred_element_type=jnp.float32) * SCALE
                s = jnp.where(mask, s, -1e9)
                m = s.max(axis=1, keepdims=True)
                w = jnp.exp(s - m)
                w = w / w.sum(axis=1, keepdims=True)
                ctx_heads.append(jnp.dot(w, vh,
                                         preferred_element_type=jnp.float32))
            ctx = jnp.concatenate(ctx_heads, axis=1)
            out_ref[pl.ds(row0, BQ), :] = jnp.dot(
                ctx, wo_ref[...], preferred_element_type=jnp.float32)

        fa = (me ^ (me >> 1)) & 1
        ga = (me >> 1) & 1
        fb = (me >> 1) & 1
        gb = me & 1

        def xchg(src_rows, n_rows, dst, dst_rows, sems, phase, dev):
            dst_ref = (dst if dst_rows is None
                       else dst.at[pl.ds(dst_rows, n_rows), :])
            return pltpu.make_async_remote_copy(
                src_ref=out_ref.at[pl.ds(src_rows, n_rows), :],
                dst_ref=dst_ref,
                send_sem=sems[0].at[phase],
                recv_sem=sems[1].at[phase],
                device_id=(dev,),
                device_id_type=pl.DeviceIdType.MESH,
            )

        sa = (sa_send, sa_recv)
        sb = (sb_send, sb_recv)

        qka = 2 * fa + ga
        qsa = 2 * fa + (1 - ga)
        qkb = 2 * fb + gb
        qsb = 2 * fb + (1 - gb)


        compute_block((2 * (1 - fa) + (1 - ga)) * QR)
        a1a = pltpu.make_async_remote_copy(
            src_ref=out_ref.at[pl.ds((2 * (1 - fa) + (1 - ga)) * QR, QR), :],
            dst_ref=c1a.at[pl.ds((1 - ga) * QR, QR), :],
            send_sem=sa_send.at[0], recv_sem=sa_recv.at[0],
            device_id=(py,), device_id_type=pl.DeviceIdType.MESH)
        a1a.start()

        compute_block(BASE_B + (2 * (1 - fb) + gb) * QR)
        b1a = pltpu.make_async_remote_copy(
            src_ref=out_ref.at[pl.ds(BASE_B + (2 * (1 - fb) + gb) * QR, QR), :],
            dst_ref=c1b.at[pl.ds(gb * QR, QR), :],
            send_sem=sb_send.at[0], recv_sem=sb_recv.at[0],
            device_id=(px,), device_id_type=pl.DeviceIdType.MESH)
        b1a.start()

        compute_block((2 * (1 - fa) + ga) * QR)
        a1b = pltpu.make_async_remote_copy(
            src_ref=out_ref.at[pl.ds((2 * (1 - fa) + ga) * QR, QR), :],
            dst_ref=c1a.at[pl.ds(ga * QR, QR), :],
            send_sem=sa_send.at[1], recv_sem=sa_recv.at[1],
            device_id=(py,), device_id_type=pl.DeviceIdType.MESH)
        a1b.start()

        compute_block(BASE_B + (2 * (1 - fb) + (1 - gb)) * QR)
        b1b = pltpu.make_async_remote_copy(
            src_ref=out_ref.at[pl.ds(BASE_B + (2 * (1 - fb) + (1 - gb)) * QR, QR), :],
            dst_ref=c1b.at[pl.ds((1 - gb) * QR, QR), :],
            send_sem=sb_send.at[1], recv_sem=sb_recv.at[1],
            device_id=(px,), device_id_type=pl.DeviceIdType.MESH)
        b1b.start()

        compute_block(qsa * QR)
        a1a.wait()
        out_ref[pl.ds(qsa * QR, QR), :] += c1a[pl.ds((1 - ga) * QR, QR), :]
        a2 = xchg(qsa * QR, QR, c2a, None, sa, 2, px)
        a2.start()

        compute_block(BASE_B + qsb * QR)
        b1a.wait()
        out_ref[pl.ds(BASE_B + qsb * QR, QR), :] += c1b[pl.ds((1 - gb) * QR, QR), :]
        b2 = xchg(BASE_B + qsb * QR, QR, c2b, None, sb, 2, py)
        b2.start()

        compute_block(qka * QR)
        a1b.wait()
        out_ref[pl.ds(qka * QR, QR), :] += c1a[pl.ds(ga * QR, QR), :]

        compute_block(BASE_B + qkb * QR)
        b1b.wait()
        out_ref[pl.ds(BASE_B + qkb * QR, QR), :] += c1b[pl.ds(gb * QR, QR), :]

        a2.wait()
        out_ref[pl.ds(qka * QR, QR), :] += c2a[...]
        a3 = xchg(qka * QR, QR, out_ref, qka * QR, sa, 3, px)
        a4k = xchg(qka * QR, QR, out_ref, qka * QR, sa, 4, py)
        a3.start(); a4k.start()

        b2.wait()
        out_ref[pl.ds(BASE_B + qkb * QR, QR), :] += c2b[...]
        b3 = xchg(BASE_B + qkb * QR, QR, out_ref, BASE_B + qkb * QR, sb, 3, py)
        b4k = xchg(BASE_B + qkb * QR, QR, out_ref, BASE_B + qkb * QR, sb, 4, px)
        b3.start(); b4k.start()

        a3.wait()
        a4s = xchg(qsa * QR, QR, out_ref, qsa * QR, sa, 5, py)
        a4s.start()
        b3.wait()
        b4s = xchg(BASE_B + qsb * QR, QR, out_ref, BASE_B + qsb * QR, sb, 5, px)
        b4s.start()

        a4k.wait(); b4k.wait(); a4s.wait(); b4s.wait()

    return pl.pallas_call(
        body,
        out_shape=jax.ShapeDtypeStruct((SQ, D_MODEL), jnp.float32),
        in_specs=[
            pl.BlockSpec(memory_space=pltpu.VMEM),
            pl.BlockSpec(memory_space=pltpu.VMEM),
            pl.BlockSpec(memory_space=pltpu.MemorySpace.HBM),
            pl.BlockSpec(memory_space=pltpu.MemorySpace.HBM),
            pl.BlockSpec(memory_space=pltpu.VMEM),
        ],
        out_specs=pl.BlockSpec(memory_space=pltpu.VMEM),
        scratch_shapes=[
            pltpu.VMEM((HQ_SH, SQ, DH), jnp.float32),
            pltpu.VMEM((HQ_SH, SQ, DH), jnp.float32),
            pltpu.SemaphoreType.DMA((2 * HQ_SH,)),
            pltpu.VMEM((HR, D_MODEL), jnp.float32),
            pltpu.VMEM((HR, D_MODEL), jnp.float32),
            pltpu.VMEM((QR, D_MODEL), jnp.float32),
            pltpu.VMEM((QR, D_MODEL), jnp.float32),
            pltpu.SemaphoreType.DMA((6,)),
            pltpu.SemaphoreType.DMA((6,)),
            pltpu.SemaphoreType.DMA((6,)),
            pltpu.SemaphoreType.DMA((6,)),
        ],
        compiler_params=pltpu.CompilerParams(
            collective_id=0,
            vmem_limit_bytes=100 * 1024 * 1024,
        ),
    )(x2, Wq, K_sh, V_sh, Wo)


def kernel(x, Wq, K_ext, V_ext, Wo):
    x2 = x.reshape(SQ, D_MODEL)
    out = _fused(x2, Wq, K_ext.reshape(SQ, 32, DH), V_ext.reshape(SQ, 32, DH),
                 Wo)
    return out.reshape(1, SQ, D_MODEL)


# device time: 101893 ns/iter; 1.3631x vs baseline; 1.0176x over previous
import jax
import jax.numpy as jnp
import numpy as np
from jax import lax
from jax.experimental import pallas as pl
from jax.experimental.pallas import tpu as pltpu

N_DEV = 4
SQ = 2048
D_MODEL = 1024
HQ_SH = 8
DH = 128
BQ = 256
NBQ = SQ // BQ
KW = 512
HR = SQ // 4
QR = SQ // 8
BASE_B = SQ // 2
SCALE = 0.08838834764831843


def _fused(x2, Wq, K_sh, V_sh, Wo):
    def body(x_ref, wq_ref, k_hbm, v_hbm, wo_ref, out_ref,
             k_ref, v_ref, kv_sems,
             c1a, c1b, c2a, c2b, sa_send, sa_recv, sb_send, sb_recv):
        me = lax.axis_index("i")
        py = me ^ 1
        px = 3 - me

        kv_copies = []
        for h in range(HQ_SH):
            for j, (src, dst) in enumerate(((k_hbm, k_ref), (v_hbm, v_ref))):
                c = pltpu.make_async_copy(
                    src.at[:, me * HQ_SH + h, :],
                    dst.at[h],
                    kv_sems.at[2 * h + j],
                )
                c.start()
                kv_copies.append(c)

        barrier = pltpu.get_barrier_semaphore()
        for nbr in (py, px):
            pl.semaphore_signal(
                barrier, inc=1, device_id=(nbr,),
                device_id_type=pl.DeviceIdType.MESH,
            )
        pl.semaphore_wait(barrier, 2)

        for c in kv_copies:
            c.wait()

        def compute_block(row0):
            start = jnp.clip(row0 - 128, 0, SQ - KW)
            q = jnp.dot(x_ref[pl.ds(row0, BQ), :], wq_ref[...],
                        preferred_element_type=jnp.float32)
            q_glob = row0 + lax.broadcasted_iota(jnp.int32, (BQ, KW), 0)
            k_glob = start + lax.broadcasted_iota(jnp.int32, (BQ, KW), 1)
            bias = jnp.where(jnp.abs(q_glob - k_glob) <= 128, 0.0, -1e9)
            ctx_heads = []
            for h in range(HQ_SH):
                qh = q[:, h * DH:(h + 1) * DH]
                kh = k_ref[h, pl.ds(start, KW), :]
                vh = v_ref[h, pl.ds(start, KW), :]
                s = lax.dot_general(
                    qh, kh, (((1,), (1,)), ((), ())),
                    preferred_element_type=jnp.float32) * SCALE
                w = jnp.exp(s + bias)
                ctx_h = jnp.dot(w, vh, preferred_element_type=jnp.float32)
                ctx_heads.append(ctx_h / w.sum(axis=1, keepdims=True))
            ctx = jnp.concatenate(ctx_heads, axis=1)
            out_ref[pl.ds(row0, BQ), :] = jnp.dot(
                ctx, wo_ref[...], preferred_element_type=jnp.float32)

        fa = (me ^ (me >> 1)) & 1
        ga = (me >> 1) & 1
        fb = (me >> 1) & 1
        gb = me & 1

        def xchg(src_rows, n_rows, dst, dst_rows, sems, phase, dev):
            dst_ref = (dst if dst_rows is None
                       else dst.at[pl.ds(dst_rows, n_rows), :])
            return pltpu.make_async_remote_copy(
                src_ref=out_ref.at[pl.ds(src_rows, n_rows), :],
                dst_ref=dst_ref,
                send_sem=sems[0].at[phase],
                recv_sem=sems[1].at[phase],
                device_id=(dev,),
                device_id_type=pl.DeviceIdType.MESH,
            )

        sa = (sa_send, sa_recv)
        sb = (sb_send, sb_recv)

        qka = 2 * fa + ga
        qsa = 2 * fa + (1 - ga)
        qkb = 2 * fb + gb
        qsb = 2 * fb + (1 - gb)


        compute_block((2 * (1 - fa) + (1 - ga)) * QR)
        a1a = pltpu.make_async_remote_copy(
            src_ref=out_ref.at[pl.ds((2 * (1 - fa) + (1 - ga)) * QR, QR), :],
            dst_ref=c1a.at[pl.ds((1 - ga) * QR, QR), :],
            send_sem=sa_send.at[0], recv_sem=sa_recv.at[0],
            device_id=(py,), device_id_type=pl.DeviceIdType.MESH)
        a1a.start()

        compute_block(BASE_B + (2 * (1 - fb) + gb) * QR)
        b1a = pltpu.make_async_remote_copy(
            src_ref=out_ref.at[pl.ds(BASE_B + (2 * (1 - fb) + gb) * QR, QR), :],
            dst_ref=c1b.at[pl.ds(gb * QR, QR), :],
            send_sem=sb_send.at[0], recv_sem=sb_recv.at[0],
            device_id=(px,), device_id_type=pl.DeviceIdType.MESH)
        b1a.start()

        compute_block((2 * (1 - fa) + ga) * QR)
        a1b = pltpu.make_async_remote_copy(
            src_ref=out_ref.at[pl.ds((2 * (1 - fa) + ga) * QR, QR), :],
            dst_ref=c1a.at[pl.ds(ga * QR, QR), :],
            send_sem=sa_send.at[1], recv_sem=sa_recv.at[1],
            device_id=(py,), device_id_type=pl.DeviceIdType.MESH)
        a1b.start()

        compute_block(BASE_B + (2 * (1 - fb) + (1 - gb)) * QR)
        b1b = pltpu.make_async_remote_copy(
            src_ref=out_ref.at[pl.ds(BASE_B + (2 * (1 - fb) + (1 - gb)) * QR, QR), :],
            dst_ref=c1b.at[pl.ds((1 - gb) * QR, QR), :],
            send_sem=sb_send.at[1], recv_sem=sb_recv.at[1],
            device_id=(px,), device_id_type=pl.DeviceIdType.MESH)
        b1b.start()

        compute_block(BASE_B + qsb * QR)
        b1a.wait()
        out_ref[pl.ds(BASE_B + qsb * QR, QR), :] += c1b[pl.ds((1 - gb) * QR, QR), :]
        b2 = xchg(BASE_B + qsb * QR, QR, c2b, None, sb, 2, py)
        b2.start()

        compute_block(qsa * QR)
        a1a.wait()
        out_ref[pl.ds(qsa * QR, QR), :] += c1a[pl.ds((1 - ga) * QR, QR), :]
        a2 = xchg(qsa * QR, QR, c2a, None, sa, 2, px)
        a2.start()

        compute_block(qka * QR)
        a1b.wait()
        out_ref[pl.ds(qka * QR, QR), :] += c1a[pl.ds(ga * QR, QR), :]

        compute_block(BASE_B + qkb * QR)
        b1b.wait()
        out_ref[pl.ds(BASE_B + qkb * QR, QR), :] += c1b[pl.ds(gb * QR, QR), :]

        a2.wait()
        out_ref[pl.ds(qka * QR, QR), :] += c2a[...]
        a3 = xchg(qka * QR, QR, out_ref, qka * QR, sa, 3, px)
        a4k = xchg(qka * QR, QR, out_ref, qka * QR, sa, 4, py)
        a3.start(); a4k.start()

        b2.wait()
        out_ref[pl.ds(BASE_B + qkb * QR, QR), :] += c2b[...]
        b3 = xchg(BASE_B + qkb * QR, QR, out_ref, BASE_B + qkb * QR, sb, 3, py)
        b4k = xchg(BASE_B + qkb * QR, QR, out_ref, BASE_B + qkb * QR, sb, 4, px)
        b3.start(); b4k.start()

        a3.wait()
        a4s = xchg(qsa * QR, QR, out_ref, qsa * QR, sa, 5, py)
        a4s.start()
        b3.wait()
        b4s = xchg(BASE_B + qsb * QR, QR, out_ref, BASE_B + qsb * QR, sb, 5, px)
        b4s.start()

        a4k.wait(); b4k.wait(); a4s.wait(); b4s.wait()

    return pl.pallas_call(
        body,
        out_shape=jax.ShapeDtypeStruct((SQ, D_MODEL), jnp.float32),
        in_specs=[
            pl.BlockSpec(memory_space=pltpu.VMEM),
            pl.BlockSpec(memory_space=pltpu.VMEM),
            pl.BlockSpec(memory_space=pltpu.MemorySpace.HBM),
            pl.BlockSpec(memory_space=pltpu.MemorySpace.HBM),
            pl.BlockSpec(memory_space=pltpu.VMEM),
        ],
        out_specs=pl.BlockSpec(memory_space=pltpu.VMEM),
        scratch_shapes=[
            pltpu.VMEM((HQ_SH, SQ, DH), jnp.float32),
            pltpu.VMEM((HQ_SH, SQ, DH), jnp.float32),
            pltpu.SemaphoreType.DMA((2 * HQ_SH,)),
            pltpu.VMEM((HR, D_MODEL), jnp.float32),
            pltpu.VMEM((HR, D_MODEL), jnp.float32),
            pltpu.VMEM((QR, D_MODEL), jnp.float32),
            pltpu.VMEM((QR, D_MODEL), jnp.float32),
            pltpu.SemaphoreType.DMA((6,)),
            pltpu.SemaphoreType.DMA((6,)),
            pltpu.SemaphoreType.DMA((6,)),
            pltpu.SemaphoreType.DMA((6,)),
        ],
        compiler_params=pltpu.CompilerParams(
            collective_id=0,
            vmem_limit_bytes=100 * 1024 * 1024,
        ),
    )(x2, Wq, K_sh, V_sh, Wo)


def kernel(x, Wq, K_ext, V_ext, Wo):
    x2 = x.reshape(SQ, D_MODEL)
    out = _fused(x2, Wq, K_ext.reshape(SQ, 32, DH), V_ext.reshape(SQ, 32, DH),
                 Wo)
    return out.reshape(1, SQ, D_MODEL)
